# baseline (speedup 1.0000x reference)
import functools

import jax
import jax.numpy as jnp
from jax import lax
from jax.experimental import pallas as pl
from jax.experimental.pallas import tpu as pltpu
from jax.experimental.pallas import tpu_sc as plsc

NC = 2
NS = 16
NW = NC * NS
CH = 80


def _matmul_call(x, w3, n, d, out):
    bn = 2000
    assert n % bn == 0

    def body(x_ref, w_ref, y_ref):
        y_ref[0] = jnp.dot(x_ref[...], w_ref[0],
                           preferred_element_type=jnp.float32)
        y_ref[1] = jnp.dot(x_ref[...], w_ref[1],
                           preferred_element_type=jnp.float32)

    return pl.pallas_call(
        body,
        grid=(n // bn,),
        in_specs=[
            pl.BlockSpec((bn, d), lambda i: (i, 0)),
            pl.BlockSpec((2, d, out), lambda i: (0, 0, 0)),
        ],
        out_specs=pl.BlockSpec((2, bn, out), lambda i: (0, i, 0)),
        out_shape=jax.ShapeDtypeStruct((2, n, out), jnp.float32),
    )(x, w3)


def _combine_call(partials, b2, n, out):
    bn = 2000
    assert n % bn == 0

    def body(p_ref, b_ref, o_ref):
        o_ref[...] = p_ref[0] + p_ref[1] + b_ref[...]

    return pl.pallas_call(
        body,
        grid=(n // bn,),
        in_specs=[
            pl.BlockSpec((2, bn, out), lambda i: (0, i, 0)),
            pl.BlockSpec((1, out), lambda i: (0, 0)),
        ],
        out_specs=pl.BlockSpec((bn, out), lambda i: (i, 0)),
        out_shape=jax.ShapeDtypeStruct((n, out), jnp.float32),
    )(partials, b2)


def _sc_edges_call(table, src2, dst2, wts2, n, out, k_chunks):
    mesh = plsc.VectorSubcoreMesh(core_axis_name="c", subcore_axis_name="s")
    rpt = -(-n // (NS * 8)) * 8
    np_ = rpt * NS

    kb_blocks = k_chunks // 8
    npairs = k_chunks // 2
    assert k_chunks % 8 == 0 and kb_blocks >= 2

    @functools.partial(
        pl.kernel,
        out_type=jax.ShapeDtypeStruct((NC, np_, out), jnp.float32),
        mesh=mesh,
        scratch_types=[
            pltpu.VMEM((2, 8, CH), jnp.int32),
            pltpu.VMEM((2, 8, CH), jnp.int32),
            pltpu.VMEM((2, 8, CH), jnp.float32),
            pltpu.VMEM((CH, out), jnp.float32),
            pltpu.VMEM((CH, out), jnp.float32),
            pltpu.VMEM((CH, out), jnp.float32),
            pltpu.VMEM((CH, out), jnp.float32),
            pltpu.VMEM_SHARED((np_, out), jnp.float32),
            pltpu.SemaphoreType.DMA,
            pltpu.SemaphoreType.DMA,
            pltpu.SemaphoreType.DMA,
            pltpu.SemaphoreType.DMA,
            pltpu.SemaphoreType.DMA,
            pltpu.SemaphoreType.DMA,
            pltpu.SemaphoreType.DMA,
            pltpu.SemaphoreType.DMA,
            pltpu.SemaphoreType.DMA,
        ],
    )
    def k(table_hbm, src_hbm, dst_hbm, w_hbm, out_hbm,
          sidx, didx, wbuf, gb0, gb1, gb2, gb3, acc,
          gsem0, gsem1, gsem2, gsem3,
          ssem0, ssem1, ssem2, ssem3, isem):
        c = lax.axis_index("c")
        s = lax.axis_index("s")
        wid = c * NS + s

        zeros16 = jnp.zeros((16,), jnp.float32)

        def zrow(r, carry):
            for h in range(out // 16):
                gb0[r, pl.ds(h * 16, 16)] = zeros16
            return carry

        lax.fori_loop(0, CH, zrow, 0)

        row0 = s * rpt
        left = rpt
        off = 0
        while left > 0:
            step = min(left, CH)
            pltpu.sync_copy(gb0.at[pl.ds(0, step)],
                            acc.at[pl.ds(row0 + off, step)])
            off += step
            left -= step

        pltpu.sync_copy(src_hbm.at[wid, pl.ds(0, 8)], sidx.at[0])
        pltpu.sync_copy(dst_hbm.at[wid, pl.ds(0, 8)], didx.at[0])
        pltpu.sync_copy(w_hbm.at[wid, pl.ds(0, 8)], wbuf.at[0])
        pltpu.async_copy(src_hbm.at[wid, pl.ds(8, 8)], sidx.at[1], isem)
        pltpu.async_copy(dst_hbm.at[wid, pl.ds(8, 8)], didx.at[1], isem)
        pltpu.async_copy(w_hbm.at[wid, pl.ds(8, 8)], wbuf.at[1], isem)

        plsc.subcore_barrier()

        gbs = [gb0, gb1, gb2, gb3]
        gsems = [gsem0, gsem1, gsem2, gsem3]
        ssems = [ssem0, ssem1, ssem2, ssem3]

        def scale(gb, wrow_slot, wrow_j):
            def grp(g, carry2):
                wv = wbuf[wrow_slot, wrow_j, pl.ds(g * 16, 16)]
                for l in range(16):
                    wb = jnp.broadcast_to(wv[l], (16,))
                    row = g * 16 + l
                    for h in range(out // 16):
                        sl = pl.ds(h * 16, 16)
                        gb[row, sl] = gb[row, sl] * wb
                return carry2

            lax.fori_loop(0, CH // 16, grp, 0)

        for m in range(4):
            pltpu.async_copy(table_hbm.at[sidx.at[0, m]], gbs[m],
                             gsems[m])

        nquads = k_chunks // 4

        def quad_body(i, carry):
            @pl.when(i % 2 == 1)
            def _():
                other = (((i - 1) // 2) + 1) % 2
                pltpu.make_async_copy(src_hbm.at[wid, pl.ds(0, 8)],
                                      sidx.at[other], isem).wait()
                pltpu.make_async_copy(dst_hbm.at[wid, pl.ds(0, 8)],
                                      didx.at[other], isem).wait()
                pltpu.make_async_copy(w_hbm.at[wid, pl.ds(0, 8)],
                                      wbuf.at[other], isem).wait()

            blk = (i // 2) % 2
            for m in range(4):
                j = 4 * (i % 2) + m
                if m >= 2:
                    pltpu.make_async_copy(
                        gbs[m - 2], acc.at[didx.at[blk, j - 2]],
                        ssems[m - 2]).wait()

                    @pl.when(i < nquads - 1)
                    def _():
                        blk2 = ((i + 1) // 2) % 2
                        j2 = 4 * ((i + 1) % 2) + m - 2
                        pltpu.async_copy(
                            table_hbm.at[sidx.at[blk2, j2]],
                            gbs[m - 2], gsems[m - 2])

                pltpu.make_async_copy(table_hbm.at[sidx.at[blk, j]],
                                      gbs[m], gsems[m]).wait()
                scale(gbs[m], blk, j)
                pltpu.async_copy(gbs[m], acc.at[didx.at[blk, j]],
                                 ssems[m], add=True)

            for m in range(2, 4):
                j = 4 * (i % 2) + m
                pltpu.make_async_copy(gbs[m], acc.at[didx.at[blk, j]],
                                      ssems[m]).wait()

                @pl.when(i < nquads - 1)
                def _():
                    blk2 = ((i + 1) // 2) % 2
                    j2 = 4 * ((i + 1) % 2) + m
                    pltpu.async_copy(table_hbm.at[sidx.at[blk2, j2]],
                                     gbs[m], gsems[m])

            @pl.when((i % 2 == 1) & (i < nquads - 1))
            def _():
                bnext = jnp.minimum((i + 3) // 2, kb_blocks - 1)
                slot = bnext % 2
                pltpu.async_copy(src_hbm.at[wid, pl.ds(bnext * 8, 8)],
                                 sidx.at[slot], isem)
                pltpu.async_copy(dst_hbm.at[wid, pl.ds(bnext * 8, 8)],
                                 didx.at[slot], isem)
                pltpu.async_copy(w_hbm.at[wid, pl.ds(bnext * 8, 8)],
                                 wbuf.at[slot], isem)
            return carry

        lax.fori_loop(0, nquads, quad_body, 0)

        plsc.subcore_barrier()
        pltpu.sync_copy(acc.at[pl.ds(row0, rpt)],
                        out_hbm.at[c, pl.ds(row0, rpt)])

    return k(table, src2, dst2, wts2)


def kernel(x, edge_index_hop1, edge_weight_hop1,
           edge_index_hop2, edge_weight_hop2, W, b):
    n, d = x.shape
    out = W.shape[1]
    e = edge_weight_hop1.shape[0]

    w3 = W.reshape(2, d, out)
    table = _matmul_call(x, w3, n, d, out).reshape(2 * n, out)

    e2 = 2 * e
    k_chunks = -(-e2 // (NW * CH * 8)) * 8
    ep = NW * CH * k_chunks
    pad = ep - e2
    eh = e // 2
    pad_rows = (jnp.arange(pad, dtype=jnp.int32) * 79) % n
    src = jnp.concatenate([
        edge_index_hop1[1, :eh], edge_index_hop2[1, :eh] + n,
        edge_index_hop1[1, eh:], edge_index_hop2[1, eh:] + n,
        pad_rows]).reshape(NW, k_chunks, CH)
    dst = jnp.concatenate([
        edge_index_hop1[0, :eh], edge_index_hop2[0, :eh],
        edge_index_hop1[0, eh:], edge_index_hop2[0, eh:],
        pad_rows]).reshape(NW, k_chunks, CH)
    wts = jnp.concatenate([
        edge_weight_hop1[:eh], edge_weight_hop2[:eh],
        edge_weight_hop1[eh:], edge_weight_hop2[eh:],
        jnp.zeros((pad,), jnp.float32)]).reshape(NW, k_chunks, CH)

    partials = _sc_edges_call(table, src, dst, wts, n, out, k_chunks)
    return _combine_call(partials, b.reshape(1, out), n, out)


# speedup vs baseline: 1.0070x; 1.0070x over previous
import functools

import jax
import jax.numpy as jnp
from jax import lax
from jax.experimental import pallas as pl
from jax.experimental.pallas import tpu as pltpu
from jax.experimental.pallas import tpu_sc as plsc

NC = 2
NS = 16
NW = NC * NS
CH = 80


def _matmul_call(x, w3, n, d, out):
    bn = 2000
    assert n % bn == 0

    def body(x_ref, w_ref, y_ref):
        y_ref[0] = jnp.dot(x_ref[...], w_ref[0],
                           preferred_element_type=jnp.float32)
        y_ref[1] = jnp.dot(x_ref[...], w_ref[1],
                           preferred_element_type=jnp.float32)

    return pl.pallas_call(
        body,
        grid=(n // bn,),
        in_specs=[
            pl.BlockSpec((bn, d), lambda i: (i, 0)),
            pl.BlockSpec((2, d, out), lambda i: (0, 0, 0)),
        ],
        out_specs=pl.BlockSpec((2, bn, out), lambda i: (0, i, 0)),
        out_shape=jax.ShapeDtypeStruct((2, n, out), jnp.float32),
    )(x, w3)


def _combine_call(partials, b2, n, out):
    bn = 2000
    assert n % bn == 0

    def body(p_ref, b_ref, o_ref):
        o_ref[...] = p_ref[0] + p_ref[1] + b_ref[...]

    return pl.pallas_call(
        body,
        grid=(n // bn,),
        in_specs=[
            pl.BlockSpec((2, bn, out), lambda i: (0, i, 0)),
            pl.BlockSpec((1, out), lambda i: (0, 0)),
        ],
        out_specs=pl.BlockSpec((bn, out), lambda i: (i, 0)),
        out_shape=jax.ShapeDtypeStruct((n, out), jnp.float32),
    )(partials, b2)


def _sc_edges_call(table, src2, dst2, wts2, n, out, k_chunks):
    mesh = plsc.VectorSubcoreMesh(core_axis_name="c", subcore_axis_name="s")
    rpt = -(-n // (NS * 8)) * 8
    np_ = rpt * NS

    kb_blocks = k_chunks // 8
    npairs = k_chunks // 2
    assert k_chunks % 8 == 0 and kb_blocks >= 2

    @functools.partial(
        pl.kernel,
        out_type=jax.ShapeDtypeStruct((NC, np_, out), jnp.float32),
        mesh=mesh,
        scratch_types=[
            pltpu.VMEM((2, 8, CH), jnp.int32),
            pltpu.VMEM((2, 8, CH), jnp.int32),
            pltpu.VMEM((2, 8, CH), jnp.float32),
            pltpu.VMEM((CH, out), jnp.float32),
            pltpu.VMEM((CH, out), jnp.float32),
            pltpu.VMEM((CH, out), jnp.float32),
            pltpu.VMEM((CH, out), jnp.float32),
            pltpu.VMEM_SHARED((np_, out), jnp.float32),
            pltpu.SemaphoreType.DMA,
            pltpu.SemaphoreType.DMA,
            pltpu.SemaphoreType.DMA,
            pltpu.SemaphoreType.DMA,
            pltpu.SemaphoreType.DMA,
            pltpu.SemaphoreType.DMA,
            pltpu.SemaphoreType.DMA,
            pltpu.SemaphoreType.DMA,
            pltpu.SemaphoreType.DMA,
        ],
    )
    def k(table_hbm, src_hbm, dst_hbm, w_hbm, out_hbm,
          sidx, didx, wbuf, gb0, gb1, gb2, gb3, acc,
          gsem0, gsem1, gsem2, gsem3,
          ssem0, ssem1, ssem2, ssem3, isem):
        c = lax.axis_index("c")
        s = lax.axis_index("s")
        wid = c * NS + s

        zeros16 = jnp.zeros((16,), jnp.float32)

        def zrow(r, carry):
            for h in range(out // 16):
                gb0[r, pl.ds(h * 16, 16)] = zeros16
            return carry

        lax.fori_loop(0, CH, zrow, 0)

        row0 = s * rpt
        left = rpt
        off = 0
        zsteps = []
        while left > 0:
            step = min(left, CH)
            pltpu.async_copy(gb0.at[pl.ds(0, step)],
                             acc.at[pl.ds(row0 + off, step)], gsem0)
            zsteps.append((off, step))
            off += step
            left -= step

        pltpu.sync_copy(src_hbm.at[wid, pl.ds(0, 8)], sidx.at[0])
        pltpu.sync_copy(dst_hbm.at[wid, pl.ds(0, 8)], didx.at[0])
        pltpu.sync_copy(w_hbm.at[wid, pl.ds(0, 8)], wbuf.at[0])
        pltpu.async_copy(src_hbm.at[wid, pl.ds(8, 8)], sidx.at[1], isem)
        pltpu.async_copy(dst_hbm.at[wid, pl.ds(8, 8)], didx.at[1], isem)
        pltpu.async_copy(w_hbm.at[wid, pl.ds(8, 8)], wbuf.at[1], isem)

        for off, step in zsteps:
            pltpu.make_async_copy(gb0.at[pl.ds(0, step)],
                                  acc.at[pl.ds(row0 + off, step)],
                                  gsem0).wait()

        plsc.subcore_barrier()

        gbs = [gb0, gb1, gb2, gb3]
        gsems = [gsem0, gsem1, gsem2, gsem3]
        ssems = [ssem0, ssem1, ssem2, ssem3]

        def scale(gb, wrow_slot, wrow_j):
            def grp(g, carry2):
                wv = wbuf[wrow_slot, wrow_j, pl.ds(g * 16, 16)]
                for l in range(16):
                    wb = jnp.broadcast_to(wv[l], (16,))
                    row = g * 16 + l
                    for h in range(out // 16):
                        sl = pl.ds(h * 16, 16)
                        gb[row, sl] = gb[row, sl] * wb
                return carry2

            lax.fori_loop(0, CH // 16, grp, 0)

        for m in range(4):
            pltpu.async_copy(table_hbm.at[sidx.at[0, m]], gbs[m],
                             gsems[m])

        nquads = k_chunks // 4

        def quad_body(i, carry):
            @pl.when(i % 2 == 1)
            def _():
                other = (((i - 1) // 2) + 1) % 2
                pltpu.make_async_copy(src_hbm.at[wid, pl.ds(0, 8)],
                                      sidx.at[other], isem).wait()
                pltpu.make_async_copy(dst_hbm.at[wid, pl.ds(0, 8)],
                                      didx.at[other], isem).wait()
                pltpu.make_async_copy(w_hbm.at[wid, pl.ds(0, 8)],
                                      wbuf.at[other], isem).wait()

            blk = (i // 2) % 2
            for m in range(4):
                j = 4 * (i % 2) + m
                if m >= 2:
                    pltpu.make_async_copy(
                        gbs[m - 2], acc.at[didx.at[blk, j - 2]],
                        ssems[m - 2]).wait()

                    @pl.when(i < nquads - 1)
                    def _():
                        blk2 = ((i + 1) // 2) % 2
                        j2 = 4 * ((i + 1) % 2) + m - 2
                        pltpu.async_copy(
                            table_hbm.at[sidx.at[blk2, j2]],
                            gbs[m - 2], gsems[m - 2])

                pltpu.make_async_copy(table_hbm.at[sidx.at[blk, j]],
                                      gbs[m], gsems[m]).wait()
                scale(gbs[m], blk, j)
                pltpu.async_copy(gbs[m], acc.at[didx.at[blk, j]],
                                 ssems[m], add=True)

            for m in range(2, 4):
                j = 4 * (i % 2) + m
                pltpu.make_async_copy(gbs[m], acc.at[didx.at[blk, j]],
                                      ssems[m]).wait()

                @pl.when(i < nquads - 1)
                def _():
                    blk2 = ((i + 1) // 2) % 2
                    j2 = 4 * ((i + 1) % 2) + m
                    pltpu.async_copy(table_hbm.at[sidx.at[blk2, j2]],
                                     gbs[m], gsems[m])

            @pl.when((i % 2 == 1) & (i < nquads - 1))
            def _():
                bnext = jnp.minimum((i + 3) // 2, kb_blocks - 1)
                slot = bnext % 2
                pltpu.async_copy(src_hbm.at[wid, pl.ds(bnext * 8, 8)],
                                 sidx.at[slot], isem)
                pltpu.async_copy(dst_hbm.at[wid, pl.ds(bnext * 8, 8)],
                                 didx.at[slot], isem)
                pltpu.async_copy(w_hbm.at[wid, pl.ds(bnext * 8, 8)],
                                 wbuf.at[slot], isem)
            return carry

        lax.fori_loop(0, nquads, quad_body, 0)

        plsc.subcore_barrier()
        pltpu.sync_copy(acc.at[pl.ds(row0, rpt)],
                        out_hbm.at[c, pl.ds(row0, rpt)])

    return k(table, src2, dst2, wts2)


def kernel(x, edge_index_hop1, edge_weight_hop1,
           edge_index_hop2, edge_weight_hop2, W, b):
    n, d = x.shape
    out = W.shape[1]
    e = edge_weight_hop1.shape[0]

    w3 = W.reshape(2, d, out)
    table = _matmul_call(x, w3, n, d, out).reshape(2 * n, out)

    e2 = 2 * e
    k_chunks = -(-e2 // (NW * CH * 8)) * 8
    ep = NW * CH * k_chunks
    pad = ep - e2
    eh = e // 2
    pad_rows = (jnp.arange(pad, dtype=jnp.int32) * 79) % n
    src = jnp.concatenate([
        edge_index_hop1[1, :eh], edge_index_hop2[1, :eh] + n,
        edge_index_hop1[1, eh:], edge_index_hop2[1, eh:] + n,
        pad_rows]).reshape(NW, k_chunks, CH)
    dst = jnp.concatenate([
        edge_index_hop1[0, :eh], edge_index_hop2[0, :eh],
        edge_index_hop1[0, eh:], edge_index_hop2[0, eh:],
        pad_rows]).reshape(NW, k_chunks, CH)
    wts = jnp.concatenate([
        edge_weight_hop1[:eh], edge_weight_hop2[:eh],
        edge_weight_hop1[eh:], edge_weight_hop2[eh:],
        jnp.zeros((pad,), jnp.float32)]).reshape(NW, k_chunks, CH)

    partials = _sc_edges_call(table, src, dst, wts, n, out, k_chunks)
    return _combine_call(partials, b.reshape(1, out), n, out)


# speedup vs baseline: 1.0559x; 1.0486x over previous
import functools

import jax
import jax.numpy as jnp
from jax import lax
from jax.experimental import pallas as pl
from jax.experimental.pallas import tpu as pltpu
from jax.experimental.pallas import tpu_sc as plsc

NC = 2
NS = 16
NW = NC * NS
CH = 80


def _matmul_call(x, w3, n, d, out):
    bn = 2000
    assert n % bn == 0

    def body(x_ref, w_ref, y_ref):
        y_ref[0] = jnp.dot(x_ref[...], w_ref[0],
                           preferred_element_type=jnp.float32)
        y_ref[1] = jnp.dot(x_ref[...], w_ref[1],
                           preferred_element_type=jnp.float32)

    return pl.pallas_call(
        body,
        grid=(n // bn,),
        in_specs=[
            pl.BlockSpec((bn, d), lambda i: (i, 0)),
            pl.BlockSpec((2, d, out), lambda i: (0, 0, 0)),
        ],
        out_specs=pl.BlockSpec((2, bn, out), lambda i: (0, i, 0)),
        out_shape=jax.ShapeDtypeStruct((2, n, out), jnp.float32),
    )(x, w3)


def _combine_call(partials, b2, n, out):
    bn = 2000
    assert n % bn == 0

    def body(p_ref, b_ref, o_ref):
        o_ref[...] = p_ref[0] + p_ref[1] + b_ref[...]

    return pl.pallas_call(
        body,
        grid=(n // bn,),
        in_specs=[
            pl.BlockSpec((2, bn, out), lambda i: (0, i, 0)),
            pl.BlockSpec((1, out), lambda i: (0, 0)),
        ],
        out_specs=pl.BlockSpec((bn, out), lambda i: (i, 0)),
        out_shape=jax.ShapeDtypeStruct((n, out), jnp.float32),
    )(partials, b2)


def _sc_edges_call(table, src2, dst2, wts2, n, out, k_chunks):
    mesh = plsc.VectorSubcoreMesh(core_axis_name="c", subcore_axis_name="s")
    rpt = -(-n // (NS * 8)) * 8
    np_ = rpt * NS

    kb_blocks = k_chunks // 8
    npairs = k_chunks // 2
    assert k_chunks % 8 == 0 and kb_blocks >= 2

    @functools.partial(
        pl.kernel,
        out_type=jax.ShapeDtypeStruct((NC, np_, out), jnp.float32),
        mesh=mesh,
        scratch_types=[
            pltpu.VMEM((2, 8, CH), jnp.int32),
            pltpu.VMEM((2, 8, CH), jnp.int32),
            pltpu.VMEM((2, 8, CH), jnp.float32),
            pltpu.VMEM((CH, out), jnp.float32),
            pltpu.VMEM((CH, out), jnp.float32),
            pltpu.VMEM((CH, out), jnp.float32),
            pltpu.VMEM((CH, out), jnp.float32),
            pltpu.VMEM_SHARED((np_, out), jnp.float32),
            pltpu.SemaphoreType.DMA,
            pltpu.SemaphoreType.DMA,
            pltpu.SemaphoreType.DMA,
            pltpu.SemaphoreType.DMA,
            pltpu.SemaphoreType.DMA,
            pltpu.SemaphoreType.DMA,
            pltpu.SemaphoreType.DMA,
            pltpu.SemaphoreType.DMA,
            pltpu.SemaphoreType.DMA,
        ],
    )
    def k(table_hbm, src_hbm, dst_hbm, w_hbm, out_hbm,
          sidx, didx, wbuf, gb0, gb1, gb2, gb3, acc,
          gsem0, gsem1, gsem2, gsem3,
          ssem0, ssem1, ssem2, ssem3, isem):
        c = lax.axis_index("c")
        s = lax.axis_index("s")
        wid = c * NS + s

        zeros16 = jnp.zeros((16,), jnp.float32)

        def zrow(r, carry):
            for h in range(out // 16):
                gb0[r, pl.ds(h * 16, 16)] = zeros16
            return carry

        lax.fori_loop(0, CH, zrow, 0)

        row0 = s * rpt
        left = rpt
        off = 0
        zsteps = []
        while left > 0:
            step = min(left, CH)
            pltpu.async_copy(gb0.at[pl.ds(0, step)],
                             acc.at[pl.ds(row0 + off, step)], gsem0)
            zsteps.append((off, step))
            off += step
            left -= step

        pltpu.sync_copy(src_hbm.at[wid, pl.ds(0, 8)], sidx.at[0])
        pltpu.sync_copy(dst_hbm.at[wid, pl.ds(0, 8)], didx.at[0])
        pltpu.sync_copy(w_hbm.at[wid, pl.ds(0, 8)], wbuf.at[0])
        pltpu.async_copy(src_hbm.at[wid, pl.ds(8, 8)], sidx.at[1], isem)
        pltpu.async_copy(dst_hbm.at[wid, pl.ds(8, 8)], didx.at[1], isem)
        pltpu.async_copy(w_hbm.at[wid, pl.ds(8, 8)], wbuf.at[1], isem)

        for off, step in zsteps:
            pltpu.make_async_copy(gb0.at[pl.ds(0, step)],
                                  acc.at[pl.ds(row0 + off, step)],
                                  gsem0).wait()

        plsc.subcore_barrier()

        gbs = [gb0, gb1, gb2, gb3]
        gsems = [gsem0, gsem1, gsem2, gsem3]
        ssems = [ssem0, ssem1, ssem2, ssem3]

        def scale(gb, wrow_slot, wrow_j):
            def grp(g, carry2):
                wv = wbuf[wrow_slot, wrow_j, pl.ds(g * 16, 16)]
                for l in range(16):
                    wb = jnp.broadcast_to(wv[l], (16,))
                    row = g * 16 + l
                    for h in range(out // 16):
                        sl = pl.ds(h * 16, 16)
                        gb[row, sl] = gb[row, sl] * wb
                return carry2

            lax.fori_loop(0, CH // 16, grp, 0)

        for m in range(4):
            pltpu.async_copy(table_hbm.at[sidx.at[0, m]], gbs[m],
                             gsems[m])

        nquads = k_chunks // 4

        def quad_body(i, carry):
            @pl.when(i % 2 == 1)
            def _():
                other = (((i - 1) // 2) + 1) % 2
                pltpu.make_async_copy(src_hbm.at[wid, pl.ds(0, 8)],
                                      sidx.at[other], isem).wait()
                pltpu.make_async_copy(dst_hbm.at[wid, pl.ds(0, 8)],
                                      didx.at[other], isem).wait()
                pltpu.make_async_copy(w_hbm.at[wid, pl.ds(0, 8)],
                                      wbuf.at[other], isem).wait()

            blk = (i // 2) % 2
            for m in range(4):
                j = 4 * (i % 2) + m
                if m >= 2:
                    pltpu.make_async_copy(
                        gbs[m - 2], acc.at[didx.at[blk, j - 2]],
                        ssems[m - 2]).wait()

                    @pl.when(i < nquads - 1)
                    def _():
                        blk2 = ((i + 1) // 2) % 2
                        j2 = 4 * ((i + 1) % 2) + m - 2
                        pltpu.async_copy(
                            table_hbm.at[sidx.at[blk2, j2]],
                            gbs[m - 2], gsems[m - 2])

                pltpu.make_async_copy(table_hbm.at[sidx.at[blk, j]],
                                      gbs[m], gsems[m]).wait()
                scale(gbs[m], blk, j)
                pltpu.async_copy(gbs[m], acc.at[didx.at[blk, j]],
                                 ssems[m], add=True)

            for m in range(2, 4):
                j = 4 * (i % 2) + m
                pltpu.make_async_copy(gbs[m], acc.at[didx.at[blk, j]],
                                      ssems[m]).wait()

                @pl.when(i < nquads - 1)
                def _():
                    blk2 = ((i + 1) // 2) % 2
                    j2 = 4 * ((i + 1) % 2) + m
                    pltpu.async_copy(table_hbm.at[sidx.at[blk2, j2]],
                                     gbs[m], gsems[m])

            @pl.when((i % 2 == 1) & (i < nquads - 1))
            def _():
                bnext = jnp.minimum((i + 3) // 2, kb_blocks - 1)
                slot = bnext % 2
                pltpu.async_copy(src_hbm.at[wid, pl.ds(bnext * 8, 8)],
                                 sidx.at[slot], isem)
                pltpu.async_copy(dst_hbm.at[wid, pl.ds(bnext * 8, 8)],
                                 didx.at[slot], isem)
                pltpu.async_copy(w_hbm.at[wid, pl.ds(bnext * 8, 8)],
                                 wbuf.at[slot], isem)
            return carry

        lax.fori_loop(0, nquads, quad_body, 0)

        plsc.subcore_barrier()
        pltpu.sync_copy(acc.at[pl.ds(row0, rpt)],
                        out_hbm.at[c, pl.ds(row0, rpt)])

    return k(table, src2, dst2, wts2)


def kernel(x, edge_index_hop1, edge_weight_hop1,
           edge_index_hop2, edge_weight_hop2, W, b):
    n, d = x.shape
    out = W.shape[1]
    e = edge_weight_hop1.shape[0]

    w3 = W.reshape(2, d, out)
    table = _matmul_call(x, w3, n, d, out).reshape(2 * n, out)

    e2 = 2 * e
    k_chunks = -(-e2 // (NW * CH * 8)) * 8
    ep = NW * CH * k_chunks
    pad = ep - e2
    eh = e // 2
    pad_rows = (jnp.arange(pad, dtype=jnp.int32) * 79) % n
    ph = pad // 2
    src = jnp.concatenate([
        edge_index_hop1[1], pad_rows[:ph],
        edge_index_hop2[1] + n, pad_rows[ph:]]).reshape(NW, k_chunks, CH)
    dst = jnp.concatenate([
        edge_index_hop1[0], pad_rows[:ph],
        edge_index_hop2[0], pad_rows[ph:]]).reshape(NW, k_chunks, CH)
    wts = jnp.concatenate([
        edge_weight_hop1, jnp.zeros((ph,), jnp.float32),
        edge_weight_hop2,
        jnp.zeros((pad - ph,), jnp.float32)]).reshape(NW, k_chunks, CH)

    partials = _sc_edges_call(table, src, dst, wts, n, out, k_chunks)
    return _combine_call(partials, b.reshape(1, out), n, out)


# speedup vs baseline: 1.0896x; 1.0319x over previous
import functools

import jax
import jax.numpy as jnp
from jax import lax
from jax.experimental import pallas as pl
from jax.experimental.pallas import tpu as pltpu
from jax.experimental.pallas import tpu_sc as plsc

NC = 2
NS = 16
NW = NC * NS
CH = 80


def _matmul_call(x, w3, n, d, out):
    bn = 2000
    assert n % bn == 0

    def body(x_ref, w_ref, y_ref):
        y_ref[0] = jnp.dot(x_ref[...], w_ref[0],
                           preferred_element_type=jnp.float32)
        y_ref[1] = jnp.dot(x_ref[...], w_ref[1],
                           preferred_element_type=jnp.float32)

    return pl.pallas_call(
        body,
        grid=(n // bn,),
        in_specs=[
            pl.BlockSpec((bn, d), lambda i: (i, 0)),
            pl.BlockSpec((2, d, out), lambda i: (0, 0, 0)),
        ],
        out_specs=pl.BlockSpec((2, bn, out), lambda i: (0, i, 0)),
        out_shape=jax.ShapeDtypeStruct((2, n, out), jnp.float32),
    )(x, w3)


def _combine_call(partials, b2, n, out):
    bn = 2000
    assert n % bn == 0

    def body(p_ref, b_ref, o_ref):
        o_ref[...] = p_ref[0] + p_ref[1] + b_ref[...]

    return pl.pallas_call(
        body,
        grid=(n // bn,),
        in_specs=[
            pl.BlockSpec((2, bn, out), lambda i: (0, i, 0)),
            pl.BlockSpec((1, out), lambda i: (0, 0)),
        ],
        out_specs=pl.BlockSpec((bn, out), lambda i: (i, 0)),
        out_shape=jax.ShapeDtypeStruct((n, out), jnp.float32),
    )(partials, b2)


def _sc_edges_call(table, src2, dst2, wts2, n, out, k_chunks):
    mesh = plsc.VectorSubcoreMesh(core_axis_name="c", subcore_axis_name="s")
    rpt = -(-n // (NS * 8)) * 8
    np_ = rpt * NS

    kb_blocks = k_chunks // 8
    npairs = k_chunks // 2
    assert k_chunks % 8 == 0 and kb_blocks >= 2

    @functools.partial(
        pl.kernel,
        out_type=jax.ShapeDtypeStruct((NC, np_, out), jnp.float32),
        mesh=mesh,
        scratch_types=[
            pltpu.VMEM((2, 8, CH), jnp.int32),
            pltpu.VMEM((2, 8, CH), jnp.int32),
            pltpu.VMEM((2, 8, CH), jnp.float32),
            pltpu.VMEM((CH, out), jnp.float32),
            pltpu.VMEM((CH, out), jnp.float32),
            pltpu.VMEM((CH, out), jnp.float32),
            pltpu.VMEM((CH, out), jnp.float32),
            pltpu.VMEM_SHARED((np_, out), jnp.float32),
            pltpu.SemaphoreType.DMA,
            pltpu.SemaphoreType.DMA,
            pltpu.SemaphoreType.DMA,
            pltpu.SemaphoreType.DMA,
            pltpu.SemaphoreType.DMA,
            pltpu.SemaphoreType.DMA,
            pltpu.SemaphoreType.DMA,
            pltpu.SemaphoreType.DMA,
            pltpu.SemaphoreType.DMA,
        ],
    )
    def k(table_hbm, src_hbm, dst_hbm, w_hbm, out_hbm,
          sidx, didx, wbuf, gb0, gb1, gb2, gb3, acc,
          gsem0, gsem1, gsem2, gsem3,
          ssem0, ssem1, ssem2, ssem3, isem):
        c = lax.axis_index("c")
        s = lax.axis_index("s")
        wid = c * NS + s

        zeros16 = jnp.zeros((16,), jnp.float32)

        def zrow(r, carry):
            for h in range(out // 16):
                gb0[r, pl.ds(h * 16, 16)] = zeros16
            return carry

        lax.fori_loop(0, CH, zrow, 0)

        row0 = s * rpt
        left = rpt
        off = 0
        zsteps = []
        while left > 0:
            step = min(left, CH)
            pltpu.async_copy(gb0.at[pl.ds(0, step)],
                             acc.at[pl.ds(row0 + off, step)], gsem0)
            zsteps.append((off, step))
            off += step
            left -= step

        pltpu.sync_copy(src_hbm.at[wid, pl.ds(0, 8)], sidx.at[0])
        pltpu.sync_copy(dst_hbm.at[wid, pl.ds(0, 8)], didx.at[0])
        pltpu.sync_copy(w_hbm.at[wid, pl.ds(0, 8)], wbuf.at[0])
        pltpu.async_copy(src_hbm.at[wid, pl.ds(8, 8)], sidx.at[1], isem)
        pltpu.async_copy(dst_hbm.at[wid, pl.ds(8, 8)], didx.at[1], isem)
        pltpu.async_copy(w_hbm.at[wid, pl.ds(8, 8)], wbuf.at[1], isem)

        for off, step in zsteps:
            pltpu.make_async_copy(gb0.at[pl.ds(0, step)],
                                  acc.at[pl.ds(row0 + off, step)],
                                  gsem0).wait()

        plsc.subcore_barrier()

        gbs = [gb0, gb1, gb2, gb3]
        gsems = [gsem0, gsem1, gsem2, gsem3]
        ssems = [ssem0, ssem1, ssem2, ssem3]

        def scale(gb, wrow_slot, wrow_j):
            def grp(g, carry2):
                wv = wbuf[wrow_slot, wrow_j, pl.ds(g * 16, 16)]
                for l in range(16):
                    wb = jnp.broadcast_to(wv[l], (16,))
                    row = g * 16 + l
                    for h in range(out // 16):
                        sl = pl.ds(h * 16, 16)
                        gb[row, sl] = gb[row, sl] * wb
                return carry2

            lax.fori_loop(0, CH // 16, grp, 0)

        for m in range(4):
            pltpu.async_copy(table_hbm.at[sidx.at[0, m]], gbs[m],
                             gsems[m])

        nquads = k_chunks // 4

        def quad_body(i, carry):
            @pl.when(i % 2 == 1)
            def _():
                other = (((i - 1) // 2) + 1) % 2
                pltpu.make_async_copy(src_hbm.at[wid, pl.ds(0, 8)],
                                      sidx.at[other], isem).wait()
                pltpu.make_async_copy(dst_hbm.at[wid, pl.ds(0, 8)],
                                      didx.at[other], isem).wait()
                pltpu.make_async_copy(w_hbm.at[wid, pl.ds(0, 8)],
                                      wbuf.at[other], isem).wait()

            blk = (i // 2) % 2
            for m in range(4):
                j = 4 * (i % 2) + m
                if m < 2:
                    @pl.when(i > 0)
                    def _():
                        jp = 4 * ((i - 1) % 2) + m + 2
                        blkp = ((i - 1) // 2) % 2
                        pltpu.make_async_copy(
                            gbs[m + 2], acc.at[didx.at[blkp, jp]],
                            ssems[m + 2]).wait()
                        pltpu.async_copy(
                            table_hbm.at[sidx.at[blk, j + 2]],
                            gbs[m + 2], gsems[m + 2])
                else:
                    pltpu.make_async_copy(
                        gbs[m - 2], acc.at[didx.at[blk, j - 2]],
                        ssems[m - 2]).wait()

                    @pl.when(i < nquads - 1)
                    def _():
                        blk2 = ((i + 1) // 2) % 2
                        j2 = 4 * ((i + 1) % 2) + m - 2
                        pltpu.async_copy(
                            table_hbm.at[sidx.at[blk2, j2]],
                            gbs[m - 2], gsems[m - 2])

                pltpu.make_async_copy(table_hbm.at[sidx.at[blk, j]],
                                      gbs[m], gsems[m]).wait()
                scale(gbs[m], blk, j)
                pltpu.async_copy(gbs[m], acc.at[didx.at[blk, j]],
                                 ssems[m], add=True)

            @pl.when((i % 2 == 1) & (i < nquads - 1))
            def _():
                bnext = jnp.minimum((i + 3) // 2, kb_blocks - 1)
                slot = bnext % 2
                pltpu.async_copy(src_hbm.at[wid, pl.ds(bnext * 8, 8)],
                                 sidx.at[slot], isem)
                pltpu.async_copy(dst_hbm.at[wid, pl.ds(bnext * 8, 8)],
                                 didx.at[slot], isem)
                pltpu.async_copy(w_hbm.at[wid, pl.ds(bnext * 8, 8)],
                                 wbuf.at[slot], isem)
            return carry

        lax.fori_loop(0, nquads, quad_body, 0)

        for m in range(2, 4):
            jl = 4 * ((nquads - 1) % 2) + m
            blkl = ((nquads - 1) // 2) % 2
            pltpu.make_async_copy(gbs[m], acc.at[didx.at[blkl, jl]],
                                  ssems[m]).wait()

        plsc.subcore_barrier()
        pltpu.sync_copy(acc.at[pl.ds(row0, rpt)],
                        out_hbm.at[c, pl.ds(row0, rpt)])

    return k(table, src2, dst2, wts2)


def kernel(x, edge_index_hop1, edge_weight_hop1,
           edge_index_hop2, edge_weight_hop2, W, b):
    n, d = x.shape
    out = W.shape[1]
    e = edge_weight_hop1.shape[0]

    w3 = W.reshape(2, d, out)
    table = _matmul_call(x, w3, n, d, out).reshape(2 * n, out)

    e2 = 2 * e
    k_chunks = -(-e2 // (NW * CH * 8)) * 8
    ep = NW * CH * k_chunks
    pad = ep - e2
    pad_rows = (jnp.arange(pad, dtype=jnp.int32) * 79) % n
    ph = pad // 2
    src = jnp.concatenate([
        edge_index_hop1[1], pad_rows[:ph],
        edge_index_hop2[1] + n, pad_rows[ph:]]).reshape(NW, k_chunks, CH)
    dst = jnp.concatenate([
        edge_index_hop1[0], pad_rows[:ph],
        edge_index_hop2[0], pad_rows[ph:]]).reshape(NW, k_chunks, CH)
    wts = jnp.concatenate([
        edge_weight_hop1, jnp.zeros((ph,), jnp.float32),
        edge_weight_hop2,
        jnp.zeros((pad - ph,), jnp.float32)]).reshape(NW, k_chunks, CH)

    partials = _sc_edges_call(table, src, dst, wts, n, out, k_chunks)
    return _combine_call(partials, b.reshape(1, out), n, out)
